# baseline (speedup 1.0000x reference)
import jax
import jax.numpy as jnp
from jax.experimental import pallas as pl

_B = 4
_S = 8193
_D = 2048
_ROWS = _B * _S
_BLOCK = 1024
_GRID = (_ROWS + _BLOCK - 1) // _BLOCK


def _onehot_body(ids_ref, out_ref):
    ids = ids_ref[...]
    iota = jax.lax.broadcasted_iota(jnp.int32, (_BLOCK, _D), 1)
    out_ref[...] = (ids == iota).astype(jnp.float32)


def kernel(input_ids):
    padded = jnp.pad(input_ids, ((0, 0), (1, 0)), constant_values=0)
    flat = padded.reshape(-1)
    flat = jnp.pad(flat, (0, _GRID * _BLOCK - _ROWS), constant_values=-1)
    ids_col = flat.reshape(_GRID * _BLOCK, 1)
    out = pl.pallas_call(
        _onehot_body,
        grid=(_GRID,),
        in_specs=[pl.BlockSpec((_BLOCK, 1), lambda i: (i, 0))],
        out_specs=pl.BlockSpec((_BLOCK, _D), lambda i: (i, 0)),
        out_shape=jax.ShapeDtypeStruct((_ROWS, _D), jnp.float32),
    )(ids_col)
    return out.reshape(_B, _S, _D)
